# baseline (speedup 1.0000x reference)
import functools

import jax
import jax.numpy as jnp
from jax import lax
from jax.experimental import pallas as pl
from jax.experimental.pallas import tpu as pltpu
from jax.experimental.pallas import tpu_sc as plsc

_NUM_CORES = 2
_NUM_SUBCORES = 16
_NUM_WORKERS = _NUM_CORES * _NUM_SUBCORES


@functools.lru_cache(maxsize=None)
def _make_gather(B, D, chunk):
    per_w = B // _NUM_WORKERS
    n_chunks = per_w // chunk
    mesh = plsc.VectorSubcoreMesh(core_axis_name="c", subcore_axis_name="s")

    @functools.partial(
        pl.kernel,
        mesh=mesh,
        out_type=jax.ShapeDtypeStruct((B, D), jnp.float32),
        scratch_types=[
            pltpu.VMEM((chunk,), jnp.int32),
            pltpu.VMEM((chunk, D), jnp.float32),
            pltpu.SemaphoreType.DMA,
        ],
        compiler_params=pltpu.CompilerParams(use_tc_tiling_on_sc=False),
    )
    def gather_kernel(idx_hbm, table_hbm, out_hbm, idx_v, rows_v, sem):
        wid = lax.axis_index("s") * _NUM_CORES + lax.axis_index("c")
        base = wid * per_w

        def body(i, carry):
            off = base + i * chunk
            pltpu.sync_copy(idx_hbm.at[pl.ds(off, chunk)], idx_v)
            pltpu.async_copy(table_hbm.at[idx_v], rows_v, sem).wait()
            pltpu.sync_copy(rows_v, out_hbm.at[pl.ds(off, chunk)])
            return carry

        lax.fori_loop(0, n_chunks, body, 0)

    return gather_kernel


def kernel(x, table):
    orig_shape = x.shape
    D = table.shape[1]
    idx = x.reshape(-1).astype(jnp.int32)
    B = idx.shape[0]
    out = _make_gather(B, D, 800)(idx, table)
    return out.reshape(*orig_shape, D)


# speedup vs baseline: 1.0236x; 1.0236x over previous
import functools

import jax
import jax.numpy as jnp
from jax import lax
from jax.experimental import pallas as pl
from jax.experimental.pallas import tpu as pltpu
from jax.experimental.pallas import tpu_sc as plsc

_NUM_CORES = 2
_NUM_SUBCORES = 16
_NUM_WORKERS = _NUM_CORES * _NUM_SUBCORES


@functools.lru_cache(maxsize=None)
def _make_gather(B, D, chunk):
    per_w = B // _NUM_WORKERS
    n_chunks = per_w // chunk
    assert per_w % chunk == 0 and n_chunks % 2 == 0
    mesh = plsc.VectorSubcoreMesh(core_axis_name="c", subcore_axis_name="s")

    @functools.partial(
        pl.kernel,
        mesh=mesh,
        out_type=jax.ShapeDtypeStruct((B, D), jnp.float32),
        scratch_types=[
            pltpu.VMEM((per_w,), jnp.int32),
            pltpu.VMEM((2, chunk, D), jnp.float32),
            pltpu.SemaphoreType.DMA,
            pltpu.SemaphoreType.DMA,
        ],
        compiler_params=pltpu.CompilerParams(use_tc_tiling_on_sc=False),
    )
    def gather_kernel(idx_hbm, table_hbm, out_hbm, idx_v, rows_v, gsem, wsem):
        wid = lax.axis_index("s") * _NUM_CORES + lax.axis_index("c")
        base = wid * per_w

        pltpu.sync_copy(idx_hbm.at[pl.ds(base, per_w)], idx_v)

        def issue_gather(cur, b):
            pltpu.async_copy(
                table_hbm.at[idx_v.at[pl.ds(cur * chunk, chunk)]],
                rows_v.at[b], gsem)

        issue_gather(0, 0)
        issue_gather(1, 1)

        def outer(g):
            for b in range(2):
                cur = g + b
                pltpu.make_async_copy(
                    table_hbm.at[idx_v.at[pl.ds(0, chunk)]],
                    rows_v.at[b], gsem).wait()
                dst = out_hbm.at[pl.ds(base + cur * chunk, chunk)]
                pltpu.async_copy(rows_v.at[b], dst, wsem)
                pltpu.make_async_copy(rows_v.at[b], dst, wsem).wait()

                @pl.when(cur + 2 < n_chunks)
                def _():
                    issue_gather(cur + 2, b)

        pl.loop(0, n_chunks, step=2)(outer)

    return gather_kernel


def kernel(x, table):
    orig_shape = x.shape
    D = table.shape[1]
    idx = x.reshape(-1).astype(jnp.int32)
    B = idx.shape[0]
    out = _make_gather(B, D, 640)(idx, table)
    return out.reshape(*orig_shape, D)
